# baseline (speedup 1.0000x reference)
import jax
import jax.numpy as jnp
from jax.experimental import pallas as pl
from jax.experimental.pallas import tpu as pltpu

_BLOCK_ROWS = 2048


def _copy_block(in_hbm, out_ref, sem):
    i = pl.program_id(0)
    src = in_hbm.at[pl.ds(i * _BLOCK_ROWS, _BLOCK_ROWS), :]
    pltpu.make_async_copy(src, out_ref, sem).start()
    pltpu.make_async_copy(src, out_ref, sem).wait()


def kernel(x, pos_emb):
    seq_len = x.shape[1]
    d_model = pos_emb.shape[1]
    num_blocks = pl.cdiv(seq_len, _BLOCK_ROWS)
    return pl.pallas_call(
        _copy_block,
        grid=(num_blocks,),
        in_specs=[pl.BlockSpec(memory_space=pl.ANY)],
        out_specs=pl.BlockSpec((_BLOCK_ROWS, d_model), lambda i: (i, 0)),
        out_shape=jax.ShapeDtypeStruct((seq_len, d_model), pos_emb.dtype),
        scratch_shapes=[pltpu.SemaphoreType.DMA],
    )(pos_emb)


# speedup vs baseline: 1.3026x; 1.3026x over previous
import jax
import jax.numpy as jnp
from jax.experimental import pallas as pl


def _copy_block(in_ref, out_ref):
    out_ref[...] = in_ref[...]


def kernel(x, pos_emb):
    seq_len = x.shape[1]
    d_model = pos_emb.shape[1]
    block_rows = 3744
    num_blocks = pl.cdiv(seq_len, block_rows)
    return pl.pallas_call(
        _copy_block,
        grid=(num_blocks,),
        in_specs=[pl.BlockSpec((block_rows, d_model), lambda i: (i, 0))],
        out_specs=pl.BlockSpec((block_rows, d_model), lambda i: (i, 0)),
        out_shape=jax.ShapeDtypeStruct((seq_len, d_model), pos_emb.dtype),
    )(pos_emb)
